# baseline (speedup 1.0000x reference)
import functools

import jax
import jax.numpy as jnp
from jax import lax
from jax.experimental import pallas as pl
from jax.experimental.pallas import tpu as pltpu
from jax.experimental.pallas import tpu_sc as plsc

C = 8
V = 1024
H = 2048
B = 4
S = 2048
SCALE = 10.0

NW = 32
TOKENS = S * B
T_PER_W = TOKENS // NW
K = 2
NBUF = 2
NCHUNK = T_PER_W // K
IDX_ROWS = T_PER_W * C // 16


def _body(table_hbm, idx_hbm, out_hbm,
          idx_v, gbufs, obufs, gsems, osems):
  nc = 2
  wid = lax.axis_index("s") * nc + lax.axis_index("c")
  row0 = wid * IDX_ROWS
  tok0 = wid * T_PER_W

  pltpu.sync_copy(idx_hbm.at[pl.ds(row0, IDX_ROWS)], idx_v)
  offs = (lax.iota(jnp.int32, 16) & 7) * V

  @pl.loop(0, IDX_ROWS)
  def _(r):
    idx_v[r] = idx_v[r] + offs

  def start_gather(chunk, b):
    pltpu.async_copy(table_hbm.at[idx_v.at[chunk]], gbufs.at[b], gsems.at[b])

  def wait_gather(b):
    pltpu.make_async_copy(
        table_hbm.at[idx_v.at[0]], gbufs.at[b], gsems.at[b]).wait()

  for b in range(NBUF):
    start_gather(b, b)

  @pl.loop(0, NCHUNK, step=NBUF)
  def _(g):
    rb = lax.shift_right_logical(g, 1) & 1
    orow = obufs.at[rb]
    @pl.when(g >= 2 * NBUF)
    def _():
      pltpu.make_async_copy(orow, out_hbm.at[0], osems.at[rb]).wait()

    for b in range(NBUF):
      gc = g + b
      wait_gather(b)
      gbuf = gbufs.at[b]

      @plsc.parallel_loop(0, H, 16, unroll=4)
      def _(j):
        col = pl.ds(j, 16)
        for k in range(K):
          v = [gbuf[k * C + c, col] for c in range(C)]
          s01 = v[0] + v[1]
          s23 = v[2] + v[3]
          s45 = v[4] + v[5]
          s67 = v[6] + v[7]
          orow[b * K + k, col] = ((s01 + s23) + (s45 + s67)) * SCALE

      @pl.when(gc + NBUF < NCHUNK)
      def _():
        start_gather(gc + NBUF, b)

    row = lax.shift_right_logical(tok0 + g * K, 2)
    pltpu.async_copy(orow, out_hbm.at[row], osems.at[rb])

  for rb in range(2):
    pltpu.make_async_copy(obufs.at[rb], out_hbm.at[0], osems.at[rb]).wait()


@jax.jit
def _run(table_flat, idx2d):
  mesh = plsc.VectorSubcoreMesh(core_axis_name="c", subcore_axis_name="s")
  return pl.kernel(
      _body,
      out_type=jax.ShapeDtypeStruct((S, B, H), jnp.float32),
      mesh=mesh,
      scratch_types=[
          pltpu.VMEM((IDX_ROWS, 16), jnp.int32),
          pltpu.VMEM((NBUF, K * C, H), jnp.float32),
          pltpu.VMEM((2, B, H), jnp.float32),
          pltpu.SemaphoreType.DMA((NBUF,)),
          pltpu.SemaphoreType.DMA((NBUF,)),
      ],
  )(table_flat, idx2d)


def kernel(audio_ids, tables):
  ids = jnp.transpose(audio_ids, (1, 0, 2))
  idx2d = ids.reshape(TOKENS * C // 16, 16)
  table_flat = tables.reshape(C * V, H)
  return _run(table_flat, idx2d)
